# baseline (speedup 1.0000x reference)
import functools

import jax
import jax.numpy as jnp
from jax import lax
from jax.experimental import pallas as pl
from jax.experimental.pallas import tpu as pltpu
from jax.experimental.pallas import tpu_sc as plsc

B = 16384
D = 128
NC = 2
NS = 16
NW = NC * NS
RPW = B // NW
CH = 64
NCH = RPW // CH
NBLK = CH // 16


def _dot_chunk(ubuf, ibuf, outv, xpose, chunk, off):
  @plsc.parallel_loop(0, NBLK, 1, unroll=2)
  def blk_body(blk):
    lane = lax.iota(jnp.int32, 16)
    blkv = jnp.full((16,), blk, jnp.int32)
    for t in range(16):
      r = off + blk * 16 + t
      prods = [ubuf[r, pl.ds(k * 16, 16)] * ibuf[r, pl.ds(k * 16, 16)]
               for k in range(D // 16)]
      while len(prods) > 1:
        prods = [prods[z] + prods[z + 1] for z in range(0, len(prods), 2)]
      xpose[blk, t, pl.ds(0, 16)] = prods[0]
    cols = [plsc.load_gather(xpose, [blkv, lane, jnp.full((16,), j, jnp.int32)])
            for j in range(16)]
    while len(cols) > 1:
      cols = [cols[z] + cols[z + 1] for z in range(0, len(cols), 2)]
    outv[pl.ds(chunk * CH + blk * 16, 16)] = cols[0]


def _mf_kernel(user_hbm, item_hbm, uf_hbm, if_hbm, out_hbm,
               uidx, iidx, ubuf, ibuf, outv, xpose,
               sem_x, su0, su1, si0, si1):
  sems_u = (su0, su1)
  sems_i = (si0, si1)
  wid = lax.axis_index("s") * NC + lax.axis_index("c")
  base = wid * RPW

  pltpu.async_copy(user_hbm.at[pl.ds(base, RPW)], uidx, sem_x)
  pltpu.async_copy(item_hbm.at[pl.ds(base, RPW)], iidx, sem_x)
  pltpu.make_async_copy(user_hbm.at[pl.ds(base, RPW)], uidx, sem_x).wait()
  pltpu.make_async_copy(item_hbm.at[pl.ds(base, RPW)], iidx, sem_x).wait()

  def start(j, slot):
    pltpu.async_copy(uf_hbm.at[uidx.at[pl.ds(j * CH, CH)]],
                     ubuf.at[pl.ds(slot * CH, CH)], sems_u[slot])
    pltpu.async_copy(if_hbm.at[iidx.at[pl.ds(j * CH, CH)]],
                     ibuf.at[pl.ds(slot * CH, CH)], sems_i[slot])

  def drain(j, slot):
    pltpu.make_async_copy(uf_hbm.at[uidx.at[pl.ds(j * CH, CH)]],
                          ubuf.at[pl.ds(slot * CH, CH)], sems_u[slot]).wait()
    pltpu.make_async_copy(if_hbm.at[iidx.at[pl.ds(j * CH, CH)]],
                          ibuf.at[pl.ds(slot * CH, CH)], sems_i[slot]).wait()

  start(0, 0)

  def chunk_body(j, _):
    nxt = j + 1
    odd = (j % 2) == 1

    @pl.when(jnp.logical_and(nxt < NCH, odd))
    def _():
      start(nxt, 0)

    @pl.when(jnp.logical_and(nxt < NCH, jnp.logical_not(odd)))
    def _():
      start(nxt, 1)

    @pl.when(odd)
    def _():
      drain(j, 1)

    @pl.when(jnp.logical_not(odd))
    def _():
      drain(j, 0)

    _dot_chunk(ubuf, ibuf, outv, xpose, j, (j % 2) * CH)
    return 0

  lax.fori_loop(0, NCH, chunk_body, 0)

  pltpu.sync_copy(outv, out_hbm.at[pl.ds(base, RPW)])


@jax.jit
def kernel(user, item, user_factors, item_factors):
  mesh = plsc.VectorSubcoreMesh(
      core_axis_name="c", subcore_axis_name="s",
      num_cores=NC, num_subcores=NS)
  return pl.kernel(
      _mf_kernel,
      out_type=jax.ShapeDtypeStruct((B,), jnp.float32),
      mesh=mesh,
      compiler_params=pltpu.CompilerParams(needs_layout_passes=False),
      scratch_types=[
          pltpu.VMEM((RPW,), jnp.int32),
          pltpu.VMEM((RPW,), jnp.int32),
          pltpu.VMEM((2 * CH, D), jnp.float32),
          pltpu.VMEM((2 * CH, D), jnp.float32),
          pltpu.VMEM((RPW,), jnp.float32),
          pltpu.VMEM((NBLK, 16, 17), jnp.float32),
          pltpu.SemaphoreType.DMA,
          pltpu.SemaphoreType.DMA,
          pltpu.SemaphoreType.DMA,
          pltpu.SemaphoreType.DMA,
          pltpu.SemaphoreType.DMA,
      ],
  )(user, item, user_factors, item_factors)


# speedup vs baseline: 1.0047x; 1.0047x over previous
import functools

import jax
import jax.numpy as jnp
from jax import lax
from jax.experimental import pallas as pl
from jax.experimental.pallas import tpu as pltpu
from jax.experimental.pallas import tpu_sc as plsc

B = 16384
D = 128
NC = 2
NS = 16
NW = NC * NS
RPW = B // NW
CH = 64
NCH = RPW // CH
NBLK = CH // 16


def _dot_chunk(ubuf, ibuf, outv, xpose, chunk, off):
  @plsc.parallel_loop(0, NBLK, 1, unroll=2)
  def blk_body(blk):
    lane = lax.iota(jnp.int32, 16)
    blkv = jnp.full((16,), blk, jnp.int32)
    for t in range(16):
      r = off + blk * 16 + t
      acc = ubuf[r, pl.ds(0, 16)] * ibuf[r, pl.ds(0, 16)]
      for k in range(1, D // 16):
        acc = acc + (ubuf[r, pl.ds(k * 16, 16)]
                     * ibuf[r, pl.ds(k * 16, 16)])
      xpose[blk, t, pl.ds(0, 16)] = acc
    acc_out = plsc.load_gather(
        xpose, [blkv, lane, jnp.zeros((16,), jnp.int32)])
    for j in range(1, 16):
      acc_out = acc_out + plsc.load_gather(
          xpose, [blkv, lane, jnp.full((16,), j, jnp.int32)])
    outv[pl.ds(chunk * CH + blk * 16, 16)] = acc_out


def _mf_kernel(user_hbm, item_hbm, uf_hbm, if_hbm, out_hbm,
               uidx, iidx, ubuf, ibuf, outv, xpose,
               sem_x, su0, su1, si0, si1, sem_o):
  sems_u = (su0, su1)
  sems_i = (si0, si1)
  wid = lax.axis_index("s") * NC + lax.axis_index("c")
  base = wid * RPW

  pltpu.async_copy(user_hbm.at[pl.ds(base, RPW)], uidx, sem_x)
  pltpu.async_copy(item_hbm.at[pl.ds(base, RPW)], iidx, sem_x)
  pltpu.make_async_copy(user_hbm.at[pl.ds(base, RPW)], uidx, sem_x).wait()
  pltpu.make_async_copy(item_hbm.at[pl.ds(base, RPW)], iidx, sem_x).wait()

  def start(j, slot):
    pltpu.async_copy(uf_hbm.at[uidx.at[pl.ds(j * CH, CH)]],
                     ubuf.at[pl.ds(slot * CH, CH)], sems_u[slot])
    pltpu.async_copy(if_hbm.at[iidx.at[pl.ds(j * CH, CH)]],
                     ibuf.at[pl.ds(slot * CH, CH)], sems_i[slot])

  def drain(j, slot):
    pltpu.make_async_copy(uf_hbm.at[uidx.at[pl.ds(j * CH, CH)]],
                          ubuf.at[pl.ds(slot * CH, CH)], sems_u[slot]).wait()
    pltpu.make_async_copy(if_hbm.at[iidx.at[pl.ds(j * CH, CH)]],
                          ibuf.at[pl.ds(slot * CH, CH)], sems_i[slot]).wait()

  start(0, 0)

  def chunk_body(j, _):
    nxt = j + 1
    odd = (j % 2) == 1

    @pl.when(jnp.logical_and(nxt < NCH, odd))
    def _():
      start(nxt, 0)

    @pl.when(jnp.logical_and(nxt < NCH, jnp.logical_not(odd)))
    def _():
      start(nxt, 1)

    @pl.when(odd)
    def _():
      drain(j, 1)

    @pl.when(jnp.logical_not(odd))
    def _():
      drain(j, 0)

    _dot_chunk(ubuf, ibuf, outv, xpose, j, (j % 2) * CH)
    pltpu.async_copy(outv.at[pl.ds(j * CH, CH)],
                     out_hbm.at[pl.ds(base + j * CH, CH)], sem_o)
    return 0

  lax.fori_loop(0, NCH, chunk_body, 0)

  for j in range(NCH):
    pltpu.make_async_copy(outv.at[pl.ds(j * CH, CH)],
                          out_hbm.at[pl.ds(base + j * CH, CH)], sem_o).wait()


@jax.jit
def kernel(user, item, user_factors, item_factors):
  mesh = plsc.VectorSubcoreMesh(
      core_axis_name="c", subcore_axis_name="s",
      num_cores=NC, num_subcores=NS)
  return pl.kernel(
      _mf_kernel,
      out_type=jax.ShapeDtypeStruct((B,), jnp.float32),
      mesh=mesh,
      compiler_params=pltpu.CompilerParams(needs_layout_passes=False),
      scratch_types=[
          pltpu.VMEM((RPW,), jnp.int32),
          pltpu.VMEM((RPW,), jnp.int32),
          pltpu.VMEM((2 * CH, D), jnp.float32),
          pltpu.VMEM((2 * CH, D), jnp.float32),
          pltpu.VMEM((RPW,), jnp.float32),
          pltpu.VMEM((NBLK, 16, 17), jnp.float32),
          pltpu.SemaphoreType.DMA,
          pltpu.SemaphoreType.DMA,
          pltpu.SemaphoreType.DMA,
          pltpu.SemaphoreType.DMA,
          pltpu.SemaphoreType.DMA,
          pltpu.SemaphoreType.DMA,
      ],
  )(user, item, user_factors, item_factors)


# speedup vs baseline: 1.0122x; 1.0075x over previous
import functools

import jax
import jax.numpy as jnp
from jax import lax
from jax.experimental import pallas as pl
from jax.experimental.pallas import tpu as pltpu
from jax.experimental.pallas import tpu_sc as plsc

B = 16384
D = 128
NC = 2
NS = 16
NW = NC * NS
RPW = B // NW
CH = 64
NCH = RPW // CH
NBLK = CH // 16


def _dot_chunk(ubuf, ibuf, outv, xpose, chunk, off):
  @plsc.parallel_loop(0, NBLK, 1, unroll=2)
  def blk_body(blk):
    lane = lax.iota(jnp.int32, 16)
    blkv = jnp.full((16,), blk, jnp.int32)
    for t in range(16):
      r = off + blk * 16 + t
      acc = ubuf[r, pl.ds(0, 16)] * ibuf[r, pl.ds(0, 16)]
      for k in range(1, D // 16):
        acc = acc + (ubuf[r, pl.ds(k * 16, 16)]
                     * ibuf[r, pl.ds(k * 16, 16)])
      xpose[blk, t, pl.ds(0, 16)] = acc
    acc_out = plsc.load_gather(
        xpose, [blkv, lane, jnp.zeros((16,), jnp.int32)])
    for j in range(1, 16):
      acc_out = acc_out + plsc.load_gather(
          xpose, [blkv, lane, jnp.full((16,), j, jnp.int32)])
    outv[pl.ds(chunk * CH + blk * 16, 16)] = acc_out


def _mf_kernel(user_hbm, item_hbm, uf_hbm, if_hbm, out_hbm,
               uidx, iidx, ubuf, ibuf, outv, xpose,
               sem_x, su0, su1, su2, si0, si1, si2, sem_o):
  sems_u = (su0, su1, su2)
  sems_i = (si0, si1, si2)
  wid = lax.axis_index("s") * NC + lax.axis_index("c")
  base = wid * RPW

  pltpu.async_copy(user_hbm.at[pl.ds(base, RPW)], uidx, sem_x)
  pltpu.async_copy(item_hbm.at[pl.ds(base, RPW)], iidx, sem_x)
  pltpu.make_async_copy(user_hbm.at[pl.ds(base, RPW)], uidx, sem_x).wait()
  pltpu.make_async_copy(item_hbm.at[pl.ds(base, RPW)], iidx, sem_x).wait()

  def start(j, slot):
    pltpu.async_copy(uf_hbm.at[uidx.at[pl.ds(j * CH, CH)]],
                     ubuf.at[pl.ds(slot * CH, CH)], sems_u[slot])
    pltpu.async_copy(if_hbm.at[iidx.at[pl.ds(j * CH, CH)]],
                     ibuf.at[pl.ds(slot * CH, CH)], sems_i[slot])

  def drain(j, slot):
    pltpu.make_async_copy(uf_hbm.at[uidx.at[pl.ds(j * CH, CH)]],
                          ubuf.at[pl.ds(slot * CH, CH)], sems_u[slot]).wait()
    pltpu.make_async_copy(if_hbm.at[iidx.at[pl.ds(j * CH, CH)]],
                          ibuf.at[pl.ds(slot * CH, CH)], sems_i[slot]).wait()

  start(0, 0)
  start(1, 1)

  def chunk_body(j, _):
    nxt = j + 2
    cur = j % 3

    for s in range(3):
      @pl.when(jnp.logical_and(nxt < NCH, (nxt % 3) == s))
      def _(s=s):
        start(nxt, s)

    for s in range(3):
      @pl.when(cur == s)
      def _(s=s):
        drain(j, s)

    _dot_chunk(ubuf, ibuf, outv, xpose, j, cur * CH)
    pltpu.async_copy(outv.at[pl.ds(j * CH, CH)],
                     out_hbm.at[pl.ds(base + j * CH, CH)], sem_o)
    return 0

  lax.fori_loop(0, NCH, chunk_body, 0)

  for j in range(NCH):
    pltpu.make_async_copy(outv.at[pl.ds(j * CH, CH)],
                          out_hbm.at[pl.ds(base + j * CH, CH)], sem_o).wait()


@jax.jit
def kernel(user, item, user_factors, item_factors):
  mesh = plsc.VectorSubcoreMesh(
      core_axis_name="c", subcore_axis_name="s",
      num_cores=NC, num_subcores=NS)
  return pl.kernel(
      _mf_kernel,
      out_type=jax.ShapeDtypeStruct((B,), jnp.float32),
      mesh=mesh,
      compiler_params=pltpu.CompilerParams(needs_layout_passes=False),
      scratch_types=[
          pltpu.VMEM((RPW,), jnp.int32),
          pltpu.VMEM((RPW,), jnp.int32),
          pltpu.VMEM((3 * CH, D), jnp.float32),
          pltpu.VMEM((3 * CH, D), jnp.float32),
          pltpu.VMEM((RPW,), jnp.float32),
          pltpu.VMEM((NBLK, 16, 17), jnp.float32),
          pltpu.SemaphoreType.DMA,
          pltpu.SemaphoreType.DMA,
          pltpu.SemaphoreType.DMA,
          pltpu.SemaphoreType.DMA,
          pltpu.SemaphoreType.DMA,
          pltpu.SemaphoreType.DMA,
          pltpu.SemaphoreType.DMA,
          pltpu.SemaphoreType.DMA,
      ],
  )(user, item, user_factors, item_factors)


# speedup vs baseline: 1.0126x; 1.0004x over previous
import jax
import jax.numpy as jnp
from jax import lax
from jax.experimental import pallas as pl
from jax.experimental.pallas import tpu as pltpu
from jax.experimental.pallas import tpu_sc as plsc

B = 16384
D = 128
NC = 2
NS = 16
NW = NC * NS
RPW = B // NW
CH = 64
NCH = RPW // CH
NBLK = CH // 16


def _dot_chunk(ubuf, ibuf, outv, xpose, chunk, off):
  @plsc.parallel_loop(0, NBLK, 1, unroll=2)
  def blk_body(blk):
    lane = lax.iota(jnp.int32, 16)
    blkv = jnp.full((16,), blk, jnp.int32)
    for t in range(16):
      r = off + blk * 16 + t
      acc = ubuf[r, pl.ds(0, 16)] * ibuf[r, pl.ds(0, 16)]
      for k in range(1, D // 16):
        acc = acc + (ubuf[r, pl.ds(k * 16, 16)]
                     * ibuf[r, pl.ds(k * 16, 16)])
      xpose[blk, t, pl.ds(0, 16)] = acc
    acc_out = plsc.load_gather(
        xpose, [blkv, lane, jnp.zeros((16,), jnp.int32)])
    for j in range(1, 16):
      acc_out = acc_out + plsc.load_gather(
          xpose, [blkv, lane, jnp.full((16,), j, jnp.int32)])
    outv[pl.ds(chunk * CH + blk * 16, 16)] = acc_out


def _mf_kernel(user_hbm, item_hbm, uf_hbm, if_hbm, out_hbm,
               uidx, iidx, ubuf, ibuf, outv, xpose,
               sem_x, su0, su1, su2, si0, si1, si2, sem_o):
  sems_u = (su0, su1, su2)
  sems_i = (si0, si1, si2)
  wid = lax.axis_index("s") * NC + lax.axis_index("c")
  base = wid * RPW

  pltpu.async_copy(user_hbm.at[pl.ds(base, RPW)], uidx, sem_x)
  pltpu.async_copy(item_hbm.at[pl.ds(base, RPW)], iidx, sem_x)
  pltpu.make_async_copy(user_hbm.at[pl.ds(base, RPW)], uidx, sem_x).wait()
  pltpu.make_async_copy(item_hbm.at[pl.ds(base, RPW)], iidx, sem_x).wait()

  def start(j, slot):
    pltpu.async_copy(uf_hbm.at[uidx.at[pl.ds(j * CH, CH)]],
                     ubuf.at[pl.ds(slot * CH, CH)], sems_u[slot])
    pltpu.async_copy(if_hbm.at[iidx.at[pl.ds(j * CH, CH)]],
                     ibuf.at[pl.ds(slot * CH, CH)], sems_i[slot])

  def drain(j, slot):
    pltpu.make_async_copy(uf_hbm.at[uidx.at[pl.ds(j * CH, CH)]],
                          ubuf.at[pl.ds(slot * CH, CH)], sems_u[slot]).wait()
    pltpu.make_async_copy(if_hbm.at[iidx.at[pl.ds(j * CH, CH)]],
                          ibuf.at[pl.ds(slot * CH, CH)], sems_i[slot]).wait()

  start(0, 0)
  start(1, 1)

  def chunk_body(j, _):
    nxt = j + 2
    cur = j % 3

    for s in range(3):
      @pl.when(jnp.logical_and(nxt < NCH, (nxt % 3) == s))
      def _(s=s):
        start(nxt, s)

    for s in range(3):
      @pl.when(cur == s)
      def _(s=s):
        drain(j, s)

    _dot_chunk(ubuf, ibuf, outv, xpose, j, cur * CH)
    pltpu.async_copy(outv.at[pl.ds(j * CH, CH)],
                     out_hbm.at[pl.ds(base + j * CH, CH)], sem_o)
    return 0

  lax.fori_loop(0, NCH, chunk_body, 0)

  for j in range(NCH):
    pltpu.make_async_copy(outv.at[pl.ds(j * CH, CH)],
                          out_hbm.at[pl.ds(base + j * CH, CH)], sem_o).wait()


@jax.jit
def kernel(user, item, user_factors, item_factors):
  mesh = plsc.VectorSubcoreMesh(
      core_axis_name="c", subcore_axis_name="s",
      num_cores=NC, num_subcores=NS)
  return pl.kernel(
      _mf_kernel,
      out_type=jax.ShapeDtypeStruct((B,), jnp.float32),
      mesh=mesh,
      compiler_params=pltpu.CompilerParams(needs_layout_passes=False),
      scratch_types=[
          pltpu.VMEM((RPW,), jnp.int32),
          pltpu.VMEM((RPW,), jnp.int32),
          pltpu.VMEM((3 * CH, D), jnp.float32),
          pltpu.VMEM((3 * CH, D), jnp.float32),
          pltpu.VMEM((RPW,), jnp.float32),
          pltpu.VMEM((NBLK, 16, 17), jnp.float32),
          pltpu.SemaphoreType.DMA,
          pltpu.SemaphoreType.DMA,
          pltpu.SemaphoreType.DMA,
          pltpu.SemaphoreType.DMA,
          pltpu.SemaphoreType.DMA,
          pltpu.SemaphoreType.DMA,
          pltpu.SemaphoreType.DMA,
          pltpu.SemaphoreType.DMA,
      ],
  )(user, item, user_factors, item_factors)
